# Initial kernel scaffold; baseline (speedup 1.0000x reference)
import jax
import jax.numpy as jnp
from jax.experimental import pallas as pl


def kernel(x, split_values, split_index_logits, estimator_weights, leaf_classes, features_by_estimator, internal_node_index, path_identifier):
    raise NotImplementedError("write your pallas kernel here")


# speedup vs baseline: 4.6477x; 4.6477x over previous
import jax
import jax.numpy as jnp
import numpy as np
from jax.experimental import pallas as pl
from jax.experimental.pallas import tpu as pltpu

B = 1024
F = 256
E = 256
S = 128
DEPTH = 6
I = 2 ** DEPTH - 1
IP = 2 ** DEPTH
L = 2 ** DEPTH

EB = 16
GRID = E // EB
R = EB * IP

_HI = jax.lax.Precision.HIGHEST


def _bitrev(v, nbits):
    r = 0
    for _ in range(nbits):
        r = (r << 1) | (v & 1)
        v >>= 1
    return r


def _node_perm():
    perm = np.zeros(I, dtype=np.int32)
    for d in range(DEPTH):
        base = 2 ** d - 1
        for q in range(2 ** d):
            perm[base + q] = base + _bitrev(q, d)
    return perm


_NODE_PERM = _node_perm()
_LEAF_PERM = np.array([_bitrev(q, DEPTH) for q in range(L)], dtype=np.int32)


def _sort_desc_sublanes(v):
    u = -v
    n = u.shape[0]
    iota = jax.lax.broadcasted_iota(jnp.int32, (n, 1), 0)
    k = 2
    while k <= n:
        j = k // 2
        while j >= 1:
            pr = jnp.concatenate([u[j:, :], u[:j, :]], axis=0)
            pd = jnp.concatenate([u[-j:, :], u[:-j, :]], axis=0)
            low = (iota & j) == 0
            part = jnp.where(low, pr, pd)
            take_min = ((iota & k) == 0) == low
            u = jnp.where(take_min, jnp.minimum(u, part), jnp.maximum(u, part))
            j //= 2
        k *= 2
    return -u


def _main_kernel(lgt_ref, svt_ref, feats_ref, ext_ref, xt_ref,
                 ye_ref, g_ref, wt_scratch):
    zt = lgt_ref[...] * 0.5
    zt = zt - jnp.max(zt, axis=0, keepdims=True)
    xs = _sort_desc_sublanes(zt)

    cat = jnp.concatenate([xs, xs * xs], axis=1)
    run = cat[0:1, :]
    rows = [run]
    for k in range(1, S):
        run = run + cat[k:k + 1, :]
        rows.append(run)
    cums = jnp.concatenate(rows, axis=0)
    c1 = cums[:, :R]
    c2 = cums[:, R:]

    iota = jax.lax.broadcasted_iota(jnp.int32, (S, 1), 0)
    rho = (iota + 1).astype(jnp.float32)
    mean = c1 / rho
    mean_sq = c2 / rho
    delta = (1.0 - rho * (mean_sq - mean * mean)) / rho
    tau = mean - jnp.sqrt(jnp.maximum(delta, 0.0))
    support = jnp.sum((tau <= xs).astype(jnp.int32), axis=0, keepdims=True)
    oh_sup = (iota == support - 1).astype(jnp.float32)
    tau_star = jnp.sum(tau * oh_sup, axis=0, keepdims=True)
    sel = jnp.square(jnp.maximum(zt - tau_star, 0.0))

    prod = sel * svt_ref[...]
    s2 = prod[0:1, :]
    for k in range(1, S):
        s2 = s2 + prod[k:k + 1, :]
    s2col = jnp.transpose(s2)

    sel_b = sel.astype(jnp.bfloat16).astype(jnp.float32)
    fiota = jax.lax.broadcasted_iota(jnp.int32, (F, 1), 0)
    for j in range(EB):
        oh = (fiota == feats_ref[j:j + 1, :]).astype(jnp.float32)
        wt_scratch[:, j * IP:(j + 1) * IP] = jax.lax.dot_general(
            oh, sel_b[:, j * IP:(j + 1) * IP], (((1,), (0,)), ((), ())),
            precision=_HI, preferred_element_type=jnp.float32)

    s1 = jax.lax.dot_general(
        wt_scratch[...], xt_ref[...], (((0,), (0,)), ((), ())),
        precision=_HI, preferred_element_type=jnp.float32)
    bits = ((s1 - s2col) > 0.0).astype(jnp.float32)

    for j in range(EB):
        base = j * IP
        p = None
        for d in range(DEPTH):
            lv = bits[base + 2 ** d - 1: base + 2 ** (d + 1) - 1, :]
            if p is None:
                p = jnp.concatenate([1.0 - lv, lv], axis=0)
            else:
                p = jnp.concatenate([p * (1.0 - lv), p * lv], axis=0)
        lc_col = ext_ref[base:base + L, 0:1]
        ew_col = ext_ref[base:base + L, 1:2]
        ye_ref[j:j + 1, :] = jnp.sum(p * lc_col, axis=0, keepdims=True)
        g_ref[j:j + 1, :] = jnp.sum(p * ew_col, axis=0, keepdims=True)


def _ensemble_kernel(g_ref, ye_ref, out_ref):
    g = g_ref[...]
    z = g * 0.5
    z = z - jnp.max(z, axis=0, keepdims=True)
    lo = jnp.full((1, z.shape[1]), -1.0, dtype=z.dtype)
    hi = jnp.zeros_like(lo)
    for _ in range(30):
        mid = 0.5 * (lo + hi)
        f = jnp.sum(jnp.square(jnp.maximum(z - mid, 0.0)), axis=0, keepdims=True)
        gt = f > 1.0
        lo = jnp.where(gt, mid, lo)
        hi = jnp.where(gt, hi, mid)
    tau0 = 0.5 * (lo + hi)
    mask = (z > tau0).astype(z.dtype)
    k = jnp.sum(mask, axis=0, keepdims=True)
    mean = jnp.sum(z * mask, axis=0, keepdims=True) / k
    mean_sq = jnp.sum(z * z * mask, axis=0, keepdims=True) / k
    delta = (1.0 - k * (mean_sq - mean * mean)) / k
    tau = mean - jnp.sqrt(jnp.maximum(delta, 0.0))
    w = jnp.square(jnp.maximum(z - tau, 0.0))
    out_ref[...] = jnp.sum(w * ye_ref[...], axis=0, keepdims=True)


@jax.jit
def _run(x, split_values, split_index_logits, estimator_weights, leaf_classes,
         features_by_estimator):
    perm = jnp.asarray(_NODE_PERM)
    lperm = jnp.asarray(_LEAF_PERM)
    lg = split_index_logits[:, perm, :]
    sv = split_values[:, perm, :]
    lgt = jnp.pad(lg, ((0, 0), (0, IP - I), (0, 0))).reshape(E * IP, S).T
    svt = jnp.pad(sv, ((0, 0), (0, IP - I), (0, 0))).reshape(E * IP, S).T
    lcr = leaf_classes[:, lperm].reshape(E * L)
    ewr = estimator_weights[:, lperm].reshape(E * L)
    ext = jnp.zeros((E * IP, 128), jnp.float32)
    ext = ext.at[:, 0].set(lcr).at[:, 1].set(ewr)
    xt = x.astype(jnp.bfloat16).astype(jnp.float32).T

    ye, g = pl.pallas_call(
        _main_kernel,
        grid=(GRID,),
        in_specs=[
            pl.BlockSpec((S, R), lambda i: (0, i)),
            pl.BlockSpec((S, R), lambda i: (0, i)),
            pl.BlockSpec((EB, S), lambda i: (i, 0)),
            pl.BlockSpec((R, 128), lambda i: (i, 0)),
            pl.BlockSpec((F, B), lambda i: (0, 0)),
        ],
        out_specs=[
            pl.BlockSpec((EB, B), lambda i: (i, 0)),
            pl.BlockSpec((EB, B), lambda i: (i, 0)),
        ],
        out_shape=[
            jax.ShapeDtypeStruct((E, B), jnp.float32),
            jax.ShapeDtypeStruct((E, B), jnp.float32),
        ],
        scratch_shapes=[pltpu.VMEM((F, R), jnp.float32)],
    )(lgt, svt, features_by_estimator, ext, xt)

    out = pl.pallas_call(
        _ensemble_kernel,
        in_specs=[
            pl.BlockSpec((E, B), lambda: (0, 0)),
            pl.BlockSpec((E, B), lambda: (0, 0)),
        ],
        out_specs=pl.BlockSpec((1, B), lambda: (0, 0)),
        out_shape=jax.ShapeDtypeStruct((1, B), jnp.float32),
    )(g, ye)
    return out.reshape(B)


def kernel(x, split_values, split_index_logits, estimator_weights,
           leaf_classes, features_by_estimator, internal_node_index,
           path_identifier):
    del internal_node_index, path_identifier
    return _run(x, split_values, split_index_logits, estimator_weights,
                leaf_classes, features_by_estimator)


# speedup vs baseline: 4.9383x; 1.0625x over previous
import functools

import jax
import jax.numpy as jnp
import numpy as np
from jax.experimental import pallas as pl
from jax.experimental.pallas import tpu as pltpu

B = 1024
F = 256
E = 256
S = 128
DEPTH = 6
I = 2 ** DEPTH - 1
IP = 2 ** DEPTH
L = 2 ** DEPTH

EB = 16
GRID = E // EB

_HI = jax.lax.Precision.HIGHEST


def _bitrev(v, nbits):
    r = 0
    for _ in range(nbits):
        r = (r << 1) | (v & 1)
        v >>= 1
    return r


def _node_perm():
    perm = np.zeros(I, dtype=np.int32)
    for d in range(DEPTH):
        base = 2 ** d - 1
        for q in range(2 ** d):
            perm[base + q] = base + _bitrev(q, d)
    return perm


_NODE_PERM = _node_perm()
_LEAF_PERM = np.array([_bitrev(q, DEPTH) for q in range(L)], dtype=np.int32)


def _entmax15_rows(z):
    z = z * 0.5
    z = z - jnp.max(z, axis=-1, keepdims=True)
    lo = jnp.full(z.shape[:-1] + (1,), -1.0, dtype=z.dtype)
    hi = jnp.zeros_like(lo)
    for _ in range(30):
        mid = 0.5 * (lo + hi)
        f = jnp.sum(jnp.square(jnp.maximum(z - mid, 0.0)), axis=-1, keepdims=True)
        gt = f > 1.0
        lo = jnp.where(gt, mid, lo)
        hi = jnp.where(gt, hi, mid)
    tau0 = 0.5 * (lo + hi)
    mask = (z > tau0).astype(z.dtype)
    k = jnp.sum(mask, axis=-1, keepdims=True)
    mean = jnp.sum(z * mask, axis=-1, keepdims=True) / k
    mean_sq = jnp.sum(z * z * mask, axis=-1, keepdims=True) / k
    delta = (1.0 - k * (mean_sq - mean * mean)) / k
    tau = mean - jnp.sqrt(jnp.maximum(delta, 0.0))
    return jnp.square(jnp.maximum(z - tau, 0.0))


def _main_kernel(logits_ref, sv_ref, feats_ref, ext_ref, xt_ref,
                 ye_ref, g_ref, w_scratch):
    sel = _entmax15_rows(logits_ref[...])
    s2 = jnp.sum(sel * sv_ref[...], axis=-1, keepdims=True)
    sel_b = sel.astype(jnp.bfloat16).astype(jnp.float32)

    fiota = jax.lax.broadcasted_iota(jnp.int32, (F, S), 0)
    for j in range(EB):
        feats_row = feats_ref[j:j + 1, :]
        oht = (fiota == feats_row).astype(jnp.float32)
        sel_j = sel_b[j * IP:(j + 1) * IP, :]
        w_scratch[j * IP:(j + 1) * IP, :] = jax.lax.dot_general(
            sel_j, oht, (((1,), (1,)), ((), ())),
            precision=_HI, preferred_element_type=jnp.float32)

    s1 = jnp.dot(w_scratch[...], xt_ref[...],
                 precision=_HI, preferred_element_type=jnp.float32)
    bits = ((s1 - s2) > 0.0).astype(jnp.float32)

    for j in range(EB):
        base = j * IP
        p = None
        for d in range(DEPTH):
            lv = bits[base + 2 ** d - 1: base + 2 ** (d + 1) - 1, :]
            if p is None:
                p = jnp.concatenate([1.0 - lv, lv], axis=0)
            else:
                p = jnp.concatenate([p * (1.0 - lv), p * lv], axis=0)
        lc_col = ext_ref[base:base + L, 0:1]
        ew_col = ext_ref[base:base + L, 1:2]
        ye_ref[j:j + 1, :] = jnp.sum(p * lc_col, axis=0, keepdims=True)
        g_ref[j:j + 1, :] = jnp.sum(p * ew_col, axis=0, keepdims=True)


def _ensemble_kernel(g_ref, ye_ref, out_ref):
    g = g_ref[...]
    z = g * 0.5
    z = z - jnp.max(z, axis=0, keepdims=True)
    lo = jnp.full((1, z.shape[1]), -1.0, dtype=z.dtype)
    hi = jnp.zeros_like(lo)
    for _ in range(30):
        mid = 0.5 * (lo + hi)
        f = jnp.sum(jnp.square(jnp.maximum(z - mid, 0.0)), axis=0, keepdims=True)
        gt = f > 1.0
        lo = jnp.where(gt, mid, lo)
        hi = jnp.where(gt, hi, mid)
    tau0 = 0.5 * (lo + hi)
    mask = (z > tau0).astype(z.dtype)
    k = jnp.sum(mask, axis=0, keepdims=True)
    mean = jnp.sum(z * mask, axis=0, keepdims=True) / k
    mean_sq = jnp.sum(z * z * mask, axis=0, keepdims=True) / k
    delta = (1.0 - k * (mean_sq - mean * mean)) / k
    tau = mean - jnp.sqrt(jnp.maximum(delta, 0.0))
    w = jnp.square(jnp.maximum(z - tau, 0.0))
    out_ref[...] = jnp.sum(w * ye_ref[...], axis=0, keepdims=True)


@jax.jit
def _run(x, split_values, split_index_logits, estimator_weights, leaf_classes,
         features_by_estimator):
    perm = jnp.asarray(_NODE_PERM)
    lperm = jnp.asarray(_LEAF_PERM)
    lg = split_index_logits[:, perm, :]
    sv = split_values[:, perm, :]
    lg = jnp.pad(lg, ((0, 0), (0, IP - I), (0, 0))).reshape(E * IP, S)
    sv = jnp.pad(sv, ((0, 0), (0, IP - I), (0, 0))).reshape(E * IP, S)
    lcr = leaf_classes[:, lperm].reshape(E * L)
    ewr = estimator_weights[:, lperm].reshape(E * L)
    ext = jnp.zeros((E * IP, 128), jnp.float32)
    ext = ext.at[:, 0].set(lcr).at[:, 1].set(ewr)
    xt = x.astype(jnp.bfloat16).astype(jnp.float32).T

    ye, g = pl.pallas_call(
        _main_kernel,
        grid=(GRID,),
        in_specs=[
            pl.BlockSpec((EB * IP, S), lambda i: (i, 0)),
            pl.BlockSpec((EB * IP, S), lambda i: (i, 0)),
            pl.BlockSpec((EB, S), lambda i: (i, 0)),
            pl.BlockSpec((EB * IP, 128), lambda i: (i, 0)),
            pl.BlockSpec((F, B), lambda i: (0, 0)),
        ],
        out_specs=[
            pl.BlockSpec((EB, B), lambda i: (i, 0)),
            pl.BlockSpec((EB, B), lambda i: (i, 0)),
        ],
        out_shape=[
            jax.ShapeDtypeStruct((E, B), jnp.float32),
            jax.ShapeDtypeStruct((E, B), jnp.float32),
        ],
        scratch_shapes=[pltpu.VMEM((EB * IP, F), jnp.float32)],
    )(lg, sv, features_by_estimator, ext, xt)

    out = pl.pallas_call(
        _ensemble_kernel,
        in_specs=[
            pl.BlockSpec((E, B), lambda: (0, 0)),
            pl.BlockSpec((E, B), lambda: (0, 0)),
        ],
        out_specs=pl.BlockSpec((1, B), lambda: (0, 0)),
        out_shape=jax.ShapeDtypeStruct((1, B), jnp.float32),
    )(g, ye)
    return out.reshape(B)


def kernel(x, split_values, split_index_logits, estimator_weights,
           leaf_classes, features_by_estimator, internal_node_index,
           path_identifier):
    del internal_node_index, path_identifier
    return _run(x, split_values, split_index_logits, estimator_weights,
                leaf_classes, features_by_estimator)
